# baseline (speedup 1.0000x reference)
import functools

import jax
import jax.numpy as jnp
from jax import lax
from jax.experimental import pallas as pl
from jax.experimental.pallas import tpu as pltpu, tpu_sc as plsc

_B, _S, _D, _H = 4, 2048, 1024, 8
_DK = _D // _H
_BS = _B * _S
_SCALE = float(_DK) ** 0.5
_R = 256
_CH = 64
_DW = _D // 2

_DNT = (((1,), (1,)), ((), ()))
_DN = (((1,), (0,)), ((), ()))


def _sc_gather_body(left_hbm, idx_hbm, out_hbm, idx_v, rows_v,
                    gs0, gs1, cs0, cs1):
    info = plsc.get_sparse_core_info()
    nc, ns, nl = info.num_cores, info.num_subcores, info.num_lanes
    rows_per = _BS // (nc * ns)
    nch = rows_per // _CH
    wid = lax.axis_index("s") * nc + lax.axis_index("c")
    base_row = wid * rows_per
    b_off = (base_row // _S) * _S
    off_vec = jnp.full((nl,), 1, jnp.int32) * b_off
    for c in range(nch):
        pltpu.sync_copy(idx_hbm.at[pl.ds(base_row + c * _CH, _CH)],
                        idx_v.at[c])
        for j in range(_CH // nl):
            sl = pl.ds(j * nl, nl)
            idx_v[c, sl] = idx_v[c, sl] + off_vec
    gsem = [gs0, gs1]
    csem = [cs0, cs1]
    gat = [pltpu.async_copy(left_hbm.at[idx_v.at[s]], rows_v.at[s], gsem[s])
           for s in range(2)]
    cout = [None, None]
    for c in range(nch):
        s = c % 2
        gat[s].wait()
        cout[s] = pltpu.async_copy(
            rows_v.at[s], out_hbm.at[pl.ds(base_row + c * _CH, _CH)], csem[s])
        if c + 2 < nch:
            cout[s].wait()
            gat[s] = pltpu.async_copy(
                left_hbm.at[idx_v.at[c + 2]], rows_v.at[s], gsem[s])
    cout[0].wait()
    cout[1].wait()


def _sc_gather(left_packed, idx):
    mesh = plsc.VectorSubcoreMesh(core_axis_name="c", subcore_axis_name="s")
    nch = _BS // (32 * _CH)
    f = functools.partial(
        pl.kernel,
        mesh=mesh,
        out_type=jax.ShapeDtypeStruct((_BS, _DW), jnp.int32),
        scratch_types=[
            pltpu.VMEM((nch, _CH), jnp.int32),
            pltpu.VMEM((2, _CH, _DW), jnp.int32),
            pltpu.SemaphoreType.DMA,
            pltpu.SemaphoreType.DMA,
            pltpu.SemaphoreType.DMA,
            pltpu.SemaphoreType.DMA,
        ],
    )(_sc_gather_body)
    return f(left_packed, idx)


def _wfuse_body(wv_ref, w1_ref, b1_ref, bv_ref, wvf_ref, bvf_ref):
    wv = wv_ref[...].astype(jnp.bfloat16)
    w1 = w1_ref[...].astype(jnp.bfloat16)
    wvf_ref[...] = lax.dot_general(
        wv, w1, _DN, preferred_element_type=jnp.float32).astype(jnp.bfloat16)
    b1b = jnp.broadcast_to(b1_ref[...], (8, _D)).astype(jnp.bfloat16)
    bv8 = lax.dot_general(b1b, wv, _DNT, preferred_element_type=jnp.float32)
    bvf_ref[...] = bv8[0:1, :] + bv_ref[...]


def _wfuse(wv, w1, b1_row, bv_row):
    return pl.pallas_call(
        _wfuse_body,
        out_shape=(
            jax.ShapeDtypeStruct((_D, _D), jnp.bfloat16),
            jax.ShapeDtypeStruct((1, _D), jnp.float32),
        ),
    )(wv, w1, b1_row, bv_row)


def _main_body(state_ref, g_ref, lng_ref, lnb_ref, wqk_ref, bq_ref,
               bk_ref, wvf_ref, bvf_ref, e_ref, et_ref, wo_ref, bo_ref,
               w2_ref, b2_ref, out_ref):
    x = state_ref[...]
    m = jnp.mean(x, axis=1, keepdims=True)
    xc = x - m
    var = jnp.mean(xc * xc, axis=1, keepdims=True)
    xn = lng_ref[...] * (xc * lax.rsqrt(var + 1e-6)) + lnb_ref[...]
    xnb = xn.astype(jnp.bfloat16)
    qk = lax.dot_general(xnb, wqk_ref[...], _DNT,
                         preferred_element_type=jnp.float32)
    q = qk[:, :_D] + bq_ref[...]
    k = qk[:, _D:] + bk_ref[...]
    v = lax.dot_general(g_ref[...], wvf_ref[...], _DNT,
                        preferred_element_type=jnp.float32) + bvf_ref[...]
    kv = k - v
    t = (q * kv).astype(jnp.bfloat16)
    s = lax.dot_general(t, e_ref[...], _DN,
                        preferred_element_type=jnp.float32)
    d = s * (1.0 / _SCALE)
    e = jnp.exp(-jnp.abs(d))
    p0 = jnp.where(d >= 0, 1.0 / (1.0 + e), e / (1.0 + e))
    pfull = lax.dot_general(p0.astype(jnp.bfloat16), et_ref[...], _DN,
                            preferred_element_type=jnp.float32)
    o = v + pfull * kv
    comb = lax.dot_general(o.astype(jnp.bfloat16), wo_ref[...], _DNT,
                           preferred_element_type=jnp.float32) + bo_ref[...]
    res = x + comb
    out_ref[...] = lax.dot_general(res.astype(jnp.bfloat16), w2_ref[...], _DNT,
                                   preferred_element_type=jnp.float32) + b2_ref[...]


def _main(state2, g2, lng, lnb, wqk, bq_r, bk_r, wvf, bvf, em, emt,
          wo_b, bo_r, w2_b, b2_r):
    row_blk = pl.BlockSpec((_R, _D), lambda i: (i, 0))
    vec = pl.BlockSpec((1, _D), lambda i: (0, 0))

    def full(a, b):
        return pl.BlockSpec((a, b), lambda i: (0, 0))

    return pl.pallas_call(
        _main_body,
        grid=(_BS // _R,),
        in_specs=[row_blk, row_blk, vec, vec, full(2 * _D, _D), vec, vec,
                  full(_D, _D), vec, full(_D, _H), full(_H, _D),
                  full(_D, _D), vec, full(_D, _D), vec],
        out_specs=row_blk,
        out_shape=jax.ShapeDtypeStruct((_BS, _D), jnp.float32),
        compiler_params=pltpu.CompilerParams(
            dimension_semantics=("parallel",)),
    )(state2, g2, lng, lnb, wqk, bq_r, bk_r, wvf, bvf, em, emt,
      wo_b, bo_r, w2_b, b2_r)


def kernel(state, left, inputad, W1, b1, W2, b2, ln_g, ln_b,
           Wq, bq, Wk, bk, Wv, bv, Wo, bo):
    bf = jnp.bfloat16
    state2 = state.reshape(_BS, _D)
    left_packed = lax.bitcast_convert_type(
        left.astype(bf).reshape(_BS, _DW, 2), jnp.int32)
    idx = inputad.reshape(_BS).astype(jnp.int32)
    wvf, bvf = _wfuse(Wv, W1, b1.reshape(1, _D), bv.reshape(1, _D))
    gp = _sc_gather(left_packed, idx)
    g2 = lax.bitcast_convert_type(gp, bf).reshape(_BS, _D)
    wqk = jnp.concatenate([Wq, Wk], axis=0).astype(bf)
    em = jnp.repeat(jnp.eye(_H, dtype=bf), _DK, axis=0)
    emt = jnp.repeat(jnp.eye(_H, dtype=bf), _DK, axis=1)
    out2 = _main(state2, g2, ln_g.reshape(1, _D), ln_b.reshape(1, _D),
                 wqk, bq.reshape(1, _D), bk.reshape(1, _D),
                 wvf, bvf, em, emt,
                 Wo.astype(bf), bo.reshape(1, _D),
                 W2.astype(bf), b2.reshape(1, _D))
    return out2.reshape(_B, _S, _D)
